# baseline (speedup 1.0000x reference)
import functools

import jax
import jax.numpy as jnp
from jax import lax
from jax.experimental import pallas as pl
from jax.experimental.pallas import tpu as pltpu
from jax.experimental.pallas import tpu_sc as plsc

_D = 256
_K = 8192
_N = 4608
_T = 128
_G = _N // _T
_COMMIT = 0.25
_DECAY = 0.99
_EPS = 1e-05


def _vq_body(x_ref, e_ref, e2_ref, dist_ref, enc_ref, idx_ref,
             counts_ref, dw_ref, lsum_ref, et_ref):
    i = pl.program_id(0)
    x = x_ref[...]
    mm = jnp.dot(x, e_ref[...], preferred_element_type=jnp.float32)
    x2 = jnp.sum(x * x, axis=1, keepdims=True)
    dist = (x2 - 2.0 * mm) + e2_ref[...]
    dist_ref[...] = dist

    mind = jnp.min(dist, axis=1, keepdims=True)
    colids = lax.broadcasted_iota(jnp.int32, dist.shape, 1)
    idx = jnp.min(jnp.where(dist == mind, colids, jnp.int32(_K)), axis=1)
    idx_ref[...] = idx.reshape(1, 1, _T)

    enc = (colids == idx[:, None]).astype(jnp.float32)
    enc_ref[...] = enc

    c = jnp.sum(enc, axis=0, keepdims=True)
    dwp = lax.dot_general(x.astype(jnp.bfloat16), enc.astype(jnp.bfloat16),
                          (((0,), (0,)), ((), ())),
                          preferred_element_type=jnp.float32)
    lpart = jnp.sum(mind)

    @pl.when(i == 0)
    def _init():
        counts_ref[...] = c
        dw_ref[...] = dwp
        lsum_ref[0, 0] = lpart
        et_ref[...] = e_ref[...].T

    @pl.when(i != 0)
    def _acc():
        counts_ref[...] += c
        dw_ref[...] += dwp
        lsum_ref[0, 0] += lpart


def _vq_core(flat_x, embeddings, e2):
    return pl.pallas_call(
        _vq_body,
        grid=(_G,),
        in_specs=[
            pl.BlockSpec((_T, _D), lambda i: (i, 0)),
            pl.BlockSpec((_D, _K), lambda i: (0, 0)),
            pl.BlockSpec((1, _K), lambda i: (0, 0)),
        ],
        out_specs=[
            pl.BlockSpec((_T, _K), lambda i: (i, 0)),
            pl.BlockSpec((_T, _K), lambda i: (i, 0)),
            pl.BlockSpec((1, 1, _T), lambda i: (i, 0, 0)),
            pl.BlockSpec((1, _K), lambda i: (0, 0)),
            pl.BlockSpec((_D, _K), lambda i: (0, 0)),
            pl.BlockSpec(memory_space=pltpu.SMEM, block_shape=(1, 1),
                         index_map=lambda i: (0, 0)),
            pl.BlockSpec((_K, _D), lambda i: (0, 0)),
        ],
        out_shape=[
            jax.ShapeDtypeStruct((_N, _K), jnp.float32),
            jax.ShapeDtypeStruct((_N, _K), jnp.float32),
            jax.ShapeDtypeStruct((_G, 1, _T), jnp.int32),
            jax.ShapeDtypeStruct((1, _K), jnp.float32),
            jax.ShapeDtypeStruct((_D, _K), jnp.float32),
            jax.ShapeDtypeStruct((1, 1), jnp.float32),
            jax.ShapeDtypeStruct((_K, _D), jnp.float32),
        ],
    )(flat_x, embeddings, e2)


_EB = 1024
_EG = _K // _EB


def _ema_body(counts_ref, csh_ref, lsum_ref, scal_ref, dw_ref, dwh_ref, e_ref,
              newe_ref, loss_ref, perp_ref, stab_ref):
    j = pl.program_id(0)
    bias = scal_ref[0, 0]
    training = scal_ref[0, 1]

    @pl.when(j == 0)
    def _scalars():
        counts = counts_ref[...]
        upd_cs = (csh_ref[...] * _DECAY + counts * (1.0 - _DECAY)) / bias
        n = jnp.sum(upd_cs)
        stab_ref[...] = (n + _K * _EPS) / ((upd_cs + _EPS) * n)
        avg = counts * (1.0 / _N)
        perp_ref[0, 0] = jnp.exp(-jnp.sum(avg * jnp.log(avg + 1e-10)))
        loss_ref[0, 0] = (_COMMIT / (_N * _D)) * lsum_ref[0, 0]

    upd_dw = (dwh_ref[...] * _DECAY + dw_ref[...] * (1.0 - _DECAY)) / bias
    inv = stab_ref[:, pl.ds(j * _EB, _EB)]
    newe_ref[...] = jnp.where(training != 0.0, upd_dw * inv, e_ref[...])


def _ema_core(counts2d, cs_hidden, lsum, scalars, dw, dw_hidden, embeddings):
    return pl.pallas_call(
        _ema_body,
        grid=(_EG,),
        in_specs=[
            pl.BlockSpec((1, _K), lambda j: (0, 0)),
            pl.BlockSpec((1, _K), lambda j: (0, 0)),
            pl.BlockSpec(memory_space=pltpu.SMEM, block_shape=(1, 1),
                         index_map=lambda j: (0, 0)),
            pl.BlockSpec(memory_space=pltpu.SMEM, block_shape=(1, 2),
                         index_map=lambda j: (0, 0)),
            pl.BlockSpec((_D, _EB), lambda j: (0, j)),
            pl.BlockSpec((_D, _EB), lambda j: (0, j)),
            pl.BlockSpec((_D, _EB), lambda j: (0, j)),
        ],
        out_specs=[
            pl.BlockSpec((_D, _EB), lambda j: (0, j)),
            pl.BlockSpec(memory_space=pltpu.SMEM, block_shape=(1, 1),
                         index_map=lambda j: (0, 0)),
            pl.BlockSpec(memory_space=pltpu.SMEM, block_shape=(1, 1),
                         index_map=lambda j: (0, 0)),
        ],
        out_shape=[
            jax.ShapeDtypeStruct((_D, _K), jnp.float32),
            jax.ShapeDtypeStruct((1, 1), jnp.float32),
            jax.ShapeDtypeStruct((1, 1), jnp.float32),
        ],
        scratch_shapes=[pltpu.VMEM((1, _K), jnp.float32)],
    )(counts2d, cs_hidden, lsum, scalars, dw, dw_hidden, embeddings)


def _make_sc_gather():
    info = plsc.get_sparse_core_info()
    nc, ns = info.num_cores, info.num_subcores
    nw = nc * ns
    bpw = _N // nw
    mesh = plsc.VectorSubcoreMesh(core_axis_name="c", subcore_axis_name="s")

    @functools.partial(
        pl.kernel, mesh=mesh,
        out_type=jax.ShapeDtypeStruct((_N, _D), jnp.float32),
        scratch_types=[
            pltpu.VMEM((bpw,), jnp.int32),
            pltpu.VMEM((bpw, _D), jnp.float32),
            pltpu.SemaphoreType.DMA,
        ],
    )
    def gather_k(table_hbm, idx_hbm, out_hbm, idx_v, rows_v, sem):
        wid = lax.axis_index("s") * nc + lax.axis_index("c")
        base = wid * bpw
        pltpu.sync_copy(idx_hbm.at[pl.ds(base, bpw)], idx_v)
        pltpu.async_copy(table_hbm.at[idx_v], rows_v, sem).wait()
        pltpu.sync_copy(rows_v, out_hbm.at[pl.ds(base, bpw)])

    return gather_k


def kernel(inputs, is_training, embeddings, ema_cluster_size_hidden,
           ema_dw_hidden, counter):
    flat_x = inputs.reshape(_N, _D)
    e2 = jnp.sum(embeddings ** 2, axis=0, keepdims=True)

    distances, encodings, idx_blocks, counts2d, dw, lsum, table = _vq_core(
        flat_x, embeddings, e2)

    idx_flat = idx_blocks.reshape(_N)
    encoding_indices = idx_flat.reshape(inputs.shape[:-1])

    gathered = _make_sc_gather()(table, idx_flat)
    quantized = gathered.reshape(inputs.shape)

    bias = 1.0 - jnp.power(_DECAY, (counter + 1).astype(jnp.float32))
    scalars = jnp.stack(
        [bias, jnp.asarray(is_training, jnp.float32)]).reshape(1, 2)
    new_embeddings, loss2d, perp2d = _ema_core(
        counts2d, ema_cluster_size_hidden.reshape(1, _K), lsum, scalars,
        dw, ema_dw_hidden, embeddings)

    return (quantized, loss2d[0, 0], perp2d[0, 0], encodings,
            encoding_indices, distances, new_embeddings)


# speedup vs baseline: 1.1818x; 1.1818x over previous
import functools

import jax
import jax.numpy as jnp
from jax import lax
from jax.experimental import pallas as pl
from jax.experimental.pallas import tpu as pltpu
from jax.experimental.pallas import tpu_sc as plsc

_D = 256
_K = 8192
_N = 4608
_T = 128
_G = _N // _T
_COMMIT = 0.25
_DECAY = 0.99
_EPS = 1e-05


def _vq_body(x_ref, e_ref, e2_ref, dist_ref, enc_ref, idx_ref,
             counts_ref, lsum_ref, et_ref):
    i = pl.program_id(0)
    x = x_ref[...]
    mm = jnp.dot(x, e_ref[...], preferred_element_type=jnp.float32)
    x2 = jnp.sum(x * x, axis=1, keepdims=True)
    dist = (x2 - 2.0 * mm) + e2_ref[...]
    dist_ref[...] = dist

    mind = jnp.min(dist, axis=1, keepdims=True)
    colids = lax.broadcasted_iota(jnp.int32, dist.shape, 1)
    idx = jnp.min(jnp.where(dist == mind, colids, jnp.int32(_K)), axis=1)
    idx_ref[...] = idx.reshape(1, 1, _T)

    enc = (colids == idx[:, None]).astype(jnp.float32)
    enc_ref[...] = enc

    c = jnp.sum(enc, axis=0, keepdims=True)
    lpart = jnp.sum(mind)

    @pl.when(i == 0)
    def _init():
        counts_ref[...] = c
        lsum_ref[0, 0] = lpart
        et_ref[...] = e_ref[...].T

    @pl.when(i != 0)
    def _acc():
        counts_ref[...] += c
        lsum_ref[0, 0] += lpart


def _vq_core(flat_x, embeddings, e2):
    return pl.pallas_call(
        _vq_body,
        grid=(_G,),
        in_specs=[
            pl.BlockSpec((_T, _D), lambda i: (i, 0)),
            pl.BlockSpec((_D, _K), lambda i: (0, 0)),
            pl.BlockSpec((1, _K), lambda i: (0, 0)),
        ],
        out_specs=[
            pl.BlockSpec((_T, _K), lambda i: (i, 0)),
            pl.BlockSpec((_T, _K), lambda i: (i, 0)),
            pl.BlockSpec((1, 1, _T), lambda i: (i, 0, 0)),
            pl.BlockSpec((1, _K), lambda i: (0, 0)),
            pl.BlockSpec(memory_space=pltpu.SMEM, block_shape=(1, 1),
                         index_map=lambda i: (0, 0)),
            pl.BlockSpec((_K, _D), lambda i: (0, 0)),
        ],
        out_shape=[
            jax.ShapeDtypeStruct((_N, _K), jnp.float32),
            jax.ShapeDtypeStruct((_N, _K), jnp.float32),
            jax.ShapeDtypeStruct((_G, 1, _T), jnp.int32),
            jax.ShapeDtypeStruct((1, _K), jnp.float32),
            jax.ShapeDtypeStruct((1, 1), jnp.float32),
            jax.ShapeDtypeStruct((_K, _D), jnp.float32),
        ],
    )(flat_x, embeddings, e2)


_EB = 1024
_EG = _K // _EB


def _ema_body(counts_ref, csh_ref, lsum_ref, scal_ref, dwt_ref, dwh_ref,
              e_ref, newe_ref, loss_ref, perp_ref, stab_ref):
    j = pl.program_id(0)
    bias = scal_ref[0, 0]
    training = scal_ref[0, 1]

    @pl.when(j == 0)
    def _scalars():
        counts = counts_ref[...]
        upd_cs = (csh_ref[...] * _DECAY + counts * (1.0 - _DECAY)) / bias
        n = jnp.sum(upd_cs)
        stab_ref[...] = (n + _K * _EPS) / ((upd_cs + _EPS) * n)
        avg = counts * (1.0 / _N)
        perp_ref[0, 0] = jnp.exp(-jnp.sum(avg * jnp.log(avg + 1e-10)))
        loss_ref[0, 0] = (_COMMIT / (_N * _D)) * lsum_ref[0, 0]

    dwb = dwt_ref[...].T
    upd_dw = (dwh_ref[...] * _DECAY + dwb * (1.0 - _DECAY)) / bias
    inv = stab_ref[:, pl.ds(j * _EB, _EB)]
    newe_ref[...] = jnp.where(training != 0.0, upd_dw * inv, e_ref[...])


def _ema_core(counts2d, cs_hidden, lsum, scalars, dwt, dw_hidden, embeddings):
    return pl.pallas_call(
        _ema_body,
        grid=(_EG,),
        in_specs=[
            pl.BlockSpec((1, _K), lambda j: (0, 0)),
            pl.BlockSpec((1, _K), lambda j: (0, 0)),
            pl.BlockSpec(memory_space=pltpu.SMEM, block_shape=(1, 1),
                         index_map=lambda j: (0, 0)),
            pl.BlockSpec(memory_space=pltpu.SMEM, block_shape=(1, 2),
                         index_map=lambda j: (0, 0)),
            pl.BlockSpec((_EB, _D), lambda j: (j, 0)),
            pl.BlockSpec((_D, _EB), lambda j: (0, j)),
            pl.BlockSpec((_D, _EB), lambda j: (0, j)),
        ],
        out_specs=[
            pl.BlockSpec((_D, _EB), lambda j: (0, j)),
            pl.BlockSpec(memory_space=pltpu.SMEM, block_shape=(1, 1),
                         index_map=lambda j: (0, 0)),
            pl.BlockSpec(memory_space=pltpu.SMEM, block_shape=(1, 1),
                         index_map=lambda j: (0, 0)),
        ],
        out_shape=[
            jax.ShapeDtypeStruct((_D, _K), jnp.float32),
            jax.ShapeDtypeStruct((1, 1), jnp.float32),
            jax.ShapeDtypeStruct((1, 1), jnp.float32),
        ],
        scratch_shapes=[pltpu.VMEM((1, _K), jnp.float32)],
    )(counts2d, cs_hidden, lsum, scalars, dwt, dw_hidden, embeddings)


def _make_sc_gather_dw():
    info = plsc.get_sparse_core_info()
    nc, ns = info.num_cores, info.num_subcores
    nw = nc * ns
    bpw = _N // nw
    tps = _N // ns
    hd = _D // nc
    rps = _K // ns
    mesh = plsc.VectorSubcoreMesh(core_axis_name="c", subcore_axis_name="s")

    gch = bpw // 2

    @functools.partial(
        pl.kernel, mesh=mesh,
        out_type=[
            jax.ShapeDtypeStruct((_N, _D), jnp.float32),
            jax.ShapeDtypeStruct((_K, _D), jnp.float32),
        ],
        scratch_types=[
            pltpu.VMEM((gch,), jnp.int32),
            pltpu.VMEM((gch, _D), jnp.float32),
            pltpu.VMEM((tps,), jnp.int32),
            pltpu.VMEM((tps, hd), jnp.float32),
            pltpu.VMEM_SHARED((_K, hd), jnp.float32),
            pltpu.SemaphoreType.DMA,
        ],
    )
    def k(table_hbm, idx_hbm, x_hbm, zeros_hbm, q_out, dwt_out,
          gidx_v, grows_v, sidx_v, xrows_v, dwt_sh, sem):
        c = lax.axis_index("c")
        s = lax.axis_index("s")
        wid = s * nc + c
        pltpu.sync_copy(zeros_hbm.at[pl.ds(s * rps, rps)],
                        dwt_sh.at[pl.ds(s * rps, rps)])
        for h in range(2):
            gbase = wid * bpw + h * gch
            pltpu.sync_copy(idx_hbm.at[pl.ds(gbase, gch)], gidx_v)
            pltpu.async_copy(table_hbm.at[gidx_v], grows_v, sem).wait()
            pltpu.sync_copy(grows_v, q_out.at[pl.ds(gbase, gch)])
        tbase = s * tps
        pltpu.sync_copy(idx_hbm.at[pl.ds(tbase, tps)], sidx_v)
        pltpu.sync_copy(
            x_hbm.at[pl.ds(tbase, tps), pl.ds(c * hd, hd)], xrows_v)
        plsc.subcore_barrier()
        pltpu.sync_copy(xrows_v, dwt_sh.at[sidx_v], add=True)
        plsc.subcore_barrier()
        pltpu.sync_copy(dwt_sh.at[pl.ds(s * rps, rps)],
                        dwt_out.at[pl.ds(s * rps, rps), pl.ds(c * hd, hd)])

    return k


def kernel(inputs, is_training, embeddings, ema_cluster_size_hidden,
           ema_dw_hidden, counter):
    flat_x = inputs.reshape(_N, _D)
    e2 = jnp.sum(embeddings ** 2, axis=0, keepdims=True)

    distances, encodings, idx_blocks, counts2d, lsum, table = _vq_core(
        flat_x, embeddings, e2)

    idx_flat = idx_blocks.reshape(_N)
    encoding_indices = idx_flat.reshape(inputs.shape[:-1])

    zeros_kd = jnp.zeros((_K, _D // 2), jnp.float32)
    gathered, dwt = _make_sc_gather_dw()(table, idx_flat, flat_x, zeros_kd)
    quantized = gathered.reshape(inputs.shape)

    bias = 1.0 - jnp.power(_DECAY, (counter + 1).astype(jnp.float32))
    scalars = jnp.stack(
        [bias, jnp.asarray(is_training, jnp.float32)]).reshape(1, 2)
    new_embeddings, loss2d, perp2d = _ema_core(
        counts2d, ema_cluster_size_hidden.reshape(1, _K), lsum, scalars,
        dwt, ema_dw_hidden, embeddings)

    return (quantized, loss2d[0, 0], perp2d[0, 0], encodings,
            encoding_indices, distances, new_embeddings)
